# Initial kernel scaffold; baseline (speedup 1.0000x reference)
import jax
import jax.numpy as jnp
from jax.experimental import pallas as pl


def kernel(x, mask):
    raise NotImplementedError("write your pallas kernel here")


# speedup vs baseline: 1.1331x; 1.1331x over previous
import functools

import jax
import jax.numpy as jnp
from jax import lax
from jax.experimental import pallas as pl
from jax.experimental.pallas import tpu as pltpu
from jax.experimental.pallas import tpu_sc as plsc

N_PTS = 131072
N_NODES = 64
N_CMP = 16
NW = 32
PTS_PER_W = N_PTS // NW
CHUNK = 512
NPAIR = PTS_PER_W // (2 * CHUNK)
XWORDS = CHUNK * N_NODES
OWORDS = CHUNK * N_CMP
MAXB = N_NODES // 16

_NEG_INF = float("-inf")


def _mask_prep(mask):
    c_ids = jnp.arange(N_CMP, dtype=jnp.int32)
    n_ids = jnp.arange(N_NODES, dtype=jnp.int32)
    eq = jnp.all(mask[:, :, None] == mask[:, None, :], axis=0)
    first = jnp.argmax(eq, axis=0).astype(jnp.int32)
    is_rep = first == c_ids
    pos = jnp.cumsum(is_rep.astype(jnp.int32)) - 1
    num_d = jnp.sum(is_rep.astype(jnp.int32))
    posmap = pos[first]
    hasany = jnp.any(mask, axis=0)
    posmap2 = jnp.where(hasany, posmap, N_CMP).astype(jnp.int32)
    rep_ids = jnp.zeros((N_CMP,), jnp.int32).at[
        jnp.where(is_rep, pos, N_CMP)
    ].set(c_ids, mode="drop")
    m_rep = mask.T[rep_ids]
    cnt = jnp.sum(m_rep.astype(jnp.int32), axis=1)
    key = jnp.where(m_rep, 0, 1) * N_NODES + n_ids[None, :]
    order = jnp.argsort(key, axis=1).astype(jnp.int32)
    jidx = jnp.minimum(n_ids[None, :], jnp.maximum(cnt[:, None] - 1, 0))
    nlist = jnp.take_along_axis(order, jidx, axis=1)
    td = jnp.maximum((cnt + 15) // 16, 1).astype(jnp.int32)
    blocks = nlist.reshape(N_CMP, MAXB, 16)
    s_ids = jnp.arange(16, dtype=jnp.int32)
    rot = (s_ids[:, None] + s_ids[None, :]) % 16
    ext2 = blocks[:, :, rot].reshape(-1)
    meta = jnp.concatenate([
        posmap2, jnp.full((16,), num_d, jnp.int32), jnp.repeat(td, 16),
    ])
    return ext2, meta


def _sc_body(x_hbm, ext_hbm, meta_hbm, out_hbm,
             xv, ov, resv, extv, metav,
             sem_in0, sem_in1, sem_out0, sem_out1):
    wid = lax.axis_index("s") * 2 + lax.axis_index("c")
    base_pt = wid * PTS_PER_W
    iota = lax.iota(jnp.int32, 16)
    iota64 = iota * N_NODES

    pltpu.sync_copy(ext_hbm, extv)
    pltpu.sync_copy(meta_hbm, metav)
    pmv = metav[pl.ds(0, 16)] * CHUNK
    num_d = metav[pl.ds(16, 16)][0]

    def _zero_g(g, c):
        resv[pl.ds(N_CMP * CHUNK + g * 16, 16)] = jnp.zeros((16,), jnp.float32)
        return c
    lax.fori_loop(0, CHUNK // 16, _zero_g, 0)

    def compute_chunk(kchunk, buf):
        xbase = buf * XWORDS

        def per_d(d, c):
            tdd = metav[pl.ds(32 + d * 16, 16)][0]

            def init_g(g, c2):
                resv[pl.ds(d * CHUNK + g * 16, 16)] = jnp.full(
                    (16,), _NEG_INF, jnp.float32)
                return c2
            lax.fori_loop(0, CHUNK // 16, init_g, 0)

            def per_block(bi, c2):
                eb = (d * MAXB + bi) * 256

                def per_g(g, c3):
                    off = d * CHUNK + g * 16
                    acc = resv[pl.ds(off, 16)]
                    idxg = xbase + g * (16 * N_NODES) + iota64
                    for s in range(16):
                        ev = extv[pl.ds(eb + s * 16, 16)]
                        vals = plsc.load_gather(xv, [idxg + ev])
                        acc = jnp.maximum(acc, vals)
                    resv[pl.ds(off, 16)] = acc
                    return c3
                lax.fori_loop(0, CHUNK // 16, per_g, 0)
                return c2
            lax.fori_loop(0, tdd, per_block, 0)
            return c
        lax.fori_loop(0, num_d, per_d, 0)

        obase = buf * OWORDS

        def per_p4(p4, c):
            for u in range(4):
                p = p4 * 4 + u
                ovec = plsc.load_gather(resv, [pmv + p])
                ov[pl.ds(obase + p * N_CMP, 16)] = ovec
            return c
        lax.fori_loop(0, CHUNK // 4, per_p4, 0)

    def in_cpy(kchunk, buf, sem):
        src = x_hbm.at[pl.ds((base_pt + kchunk * CHUNK) * N_NODES, XWORDS)]
        return pltpu.make_async_copy(src, xv.at[pl.ds(buf * XWORDS, XWORDS)], sem)

    def out_cpy(kchunk, buf, sem):
        dst = out_hbm.at[pl.ds((base_pt + kchunk * CHUNK) * N_CMP, OWORDS)]
        return pltpu.make_async_copy(ov.at[pl.ds(buf * OWORDS, OWORDS)], dst, sem)

    in_cpy(0, 0, sem_in0).start()

    def per_pair(j, c):
        k0 = 2 * j
        in_cpy(k0, 0, sem_in0).wait()
        in_cpy(k0 + 1, 1, sem_in1).start()

        @pl.when(j > 0)
        def _():
            out_cpy(k0 - 2, 0, sem_out0).wait()
        compute_chunk(k0, 0)
        out_cpy(k0, 0, sem_out0).start()

        in_cpy(k0 + 1, 1, sem_in1).wait()

        @pl.when(j < NPAIR - 1)
        def _():
            in_cpy(k0 + 2, 0, sem_in0).start()

        @pl.when(j > 0)
        def _():
            out_cpy(k0 - 1, 1, sem_out1).wait()
        compute_chunk(k0 + 1, 1)
        out_cpy(k0 + 1, 1, sem_out1).start()
        return c

    lax.fori_loop(0, NPAIR, per_pair, 0)
    out_cpy(2 * NPAIR - 2, 0, sem_out0).wait()
    out_cpy(2 * NPAIR - 1, 1, sem_out1).wait()


@functools.cache
def _sc_call():
    return functools.partial(
        pl.kernel,
        out_type=jax.ShapeDtypeStruct((N_PTS * N_CMP,), jnp.float32),
        mesh=plsc.VectorSubcoreMesh(core_axis_name="c", subcore_axis_name="s"),
        compiler_params=pltpu.CompilerParams(needs_layout_passes=False),
        scratch_types=[
            pltpu.VMEM((2 * XWORDS,), jnp.float32),
            pltpu.VMEM((2 * OWORDS,), jnp.float32),
            pltpu.VMEM(((N_CMP + 1) * CHUNK,), jnp.float32),
            pltpu.VMEM((N_CMP * MAXB * 256,), jnp.int32),
            pltpu.VMEM((288,), jnp.int32),
            pltpu.SemaphoreType.DMA,
            pltpu.SemaphoreType.DMA,
            pltpu.SemaphoreType.DMA,
            pltpu.SemaphoreType.DMA,
        ],
    )(_sc_body)


def kernel(x, mask):
    ext2, meta = _mask_prep(mask)
    out_flat = _sc_call()(x.reshape(-1), ext2, meta)
    return out_flat.reshape(N_PTS, N_CMP)


# speedup vs baseline: 1.1982x; 1.0574x over previous
import functools

import jax
import jax.numpy as jnp
from jax import lax
from jax.experimental import pallas as pl
from jax.experimental.pallas import tpu as pltpu
from jax.experimental.pallas import tpu_sc as plsc

N_PTS = 131072
N_NODES = 64
N_CMP = 16
NW = 32
PTS_PER_W = N_PTS // NW
CHUNK = 512
NPAIR = PTS_PER_W // (2 * CHUNK)
XWORDS = CHUNK * N_NODES
OWORDS = CHUNK * N_CMP
MAXB = N_NODES // 16

_NEG_INF = float("-inf")


def _mask_prep(mask):
    c_ids = jnp.arange(N_CMP, dtype=jnp.int32)
    n_ids = jnp.arange(N_NODES, dtype=jnp.int32)
    eq = jnp.all(mask[:, :, None] == mask[:, None, :], axis=0)
    first = jnp.argmax(eq, axis=0).astype(jnp.int32)
    is_rep = first == c_ids
    pos = jnp.cumsum(is_rep.astype(jnp.int32)) - 1
    num_d = jnp.sum(is_rep.astype(jnp.int32))
    posmap = pos[first]
    hasany = jnp.any(mask, axis=0)
    posmap2 = jnp.where(hasany, posmap, N_CMP).astype(jnp.int32)
    rep_ids = jnp.zeros((N_CMP,), jnp.int32).at[
        jnp.where(is_rep, pos, N_CMP)
    ].set(c_ids, mode="drop")
    m_rep = mask.T[rep_ids]
    cnt = jnp.sum(m_rep.astype(jnp.int32), axis=1)
    key = jnp.where(m_rep, 0, 1) * N_NODES + n_ids[None, :]
    order = jnp.argsort(key, axis=1).astype(jnp.int32)
    jidx = jnp.minimum(n_ids[None, :], jnp.maximum(cnt[:, None] - 1, 0))
    nlist = jnp.take_along_axis(order, jidx, axis=1)
    td = jnp.maximum((cnt + 15) // 16, 1).astype(jnp.int32)
    blocks = nlist.reshape(N_CMP, MAXB, 16)
    s_ids = jnp.arange(16, dtype=jnp.int32)
    rot = (s_ids[:, None] + s_ids[None, :]) % 16
    ext2 = blocks[:, :, rot].reshape(-1)
    meta = jnp.concatenate([
        posmap2, jnp.full((16,), num_d, jnp.int32), jnp.repeat(td, 16),
    ])
    return ext2, meta


def _sc_body(x_hbm, ext_hbm, meta_hbm, out_hbm,
             xv, ov, resv, extv, metav,
             sem_in0, sem_in1, sem_out0, sem_out1):
    wid = lax.axis_index("s") * 2 + lax.axis_index("c")
    base_pt = wid * PTS_PER_W
    iota = lax.iota(jnp.int32, 16)

    pltpu.sync_copy(ext_hbm, extv)
    pltpu.sync_copy(meta_hbm, metav)
    pmv = metav[pl.ds(0, 16)] * CHUNK
    num_d = metav[pl.ds(16, 16)][0]

    def _zero_g(g, c):
        resv[pl.ds(N_CMP * CHUNK + g * 16, 16)] = jnp.zeros((16,), jnp.float32)
        return c
    lax.fori_loop(0, CHUNK // 16, _zero_g, 0)

    def compute_chunk(kchunk, buf):
        pbase = buf * CHUNK

        def per_d(d, c):
            tdd = metav[pl.ds(32 + d * 16, 16)][0]

            def init_g(g, c2):
                resv[pl.ds(d * CHUNK + g * 16, 16)] = jnp.full(
                    (16,), _NEG_INF, jnp.float32)
                return c2
            lax.fori_loop(0, CHUNK // 16, init_g, 0)

            def per_block(bi, c2):
                eb = (d * MAXB + bi) * 256
                svs = [extv[pl.ds(eb + s * 16, 16)] for s in range(16)]

                def per_g(g, c3):
                    off = d * CHUNK + g * 32
                    acc0 = resv[pl.ds(off, 16)]
                    acc1 = resv[pl.ds(off + 16, 16)]
                    ptv0 = pbase + g * 32 + iota
                    ptv1 = ptv0 + 16
                    for s in range(16):
                        acc0 = jnp.maximum(acc0, plsc.load_gather(xv, [ptv0, svs[s]]))
                        acc1 = jnp.maximum(acc1, plsc.load_gather(xv, [ptv1, svs[s]]))
                    resv[pl.ds(off, 16)] = acc0
                    resv[pl.ds(off + 16, 16)] = acc1
                    return c3
                lax.fori_loop(0, CHUNK // 32, per_g, 0)
                return c2
            lax.fori_loop(0, tdd, per_block, 0)
            return c
        lax.fori_loop(0, num_d, per_d, 0)

        def per_p4(p4, c):
            for u in range(4):
                p = p4 * 4 + u
                ovec = plsc.load_gather(resv, [pmv + p])
                ov[pbase + p, :] = ovec
            return c
        lax.fori_loop(0, CHUNK // 4, per_p4, 0)

    def in_cpy(kchunk, buf, sem):
        src = x_hbm.at[pl.ds(base_pt + kchunk * CHUNK, CHUNK), :]
        return pltpu.make_async_copy(
            src, xv.at[pl.ds(buf * CHUNK, CHUNK), :], sem)

    def out_cpy(kchunk, buf, sem):
        dst = out_hbm.at[pl.ds(base_pt + kchunk * CHUNK, CHUNK), :]
        return pltpu.make_async_copy(
            ov.at[pl.ds(buf * CHUNK, CHUNK), :], dst, sem)

    in_cpy(0, 0, sem_in0).start()

    def per_pair(j, c):
        k0 = 2 * j
        in_cpy(k0, 0, sem_in0).wait()
        in_cpy(k0 + 1, 1, sem_in1).start()

        @pl.when(j > 0)
        def _():
            out_cpy(k0 - 2, 0, sem_out0).wait()
        compute_chunk(k0, 0)
        out_cpy(k0, 0, sem_out0).start()

        in_cpy(k0 + 1, 1, sem_in1).wait()

        @pl.when(j < NPAIR - 1)
        def _():
            in_cpy(k0 + 2, 0, sem_in0).start()

        @pl.when(j > 0)
        def _():
            out_cpy(k0 - 1, 1, sem_out1).wait()
        compute_chunk(k0 + 1, 1)
        out_cpy(k0 + 1, 1, sem_out1).start()
        return c

    lax.fori_loop(0, NPAIR, per_pair, 0)
    out_cpy(2 * NPAIR - 2, 0, sem_out0).wait()
    out_cpy(2 * NPAIR - 1, 1, sem_out1).wait()


@functools.cache
def _sc_call():
    return functools.partial(
        pl.kernel,
        out_type=jax.ShapeDtypeStruct((N_PTS, N_CMP), jnp.float32),
        mesh=plsc.VectorSubcoreMesh(core_axis_name="c", subcore_axis_name="s"),
        compiler_params=pltpu.CompilerParams(
            needs_layout_passes=False, use_tc_tiling_on_sc=False),
        scratch_types=[
            pltpu.VMEM((2 * CHUNK, N_NODES), jnp.float32),
            pltpu.VMEM((2 * CHUNK, N_CMP), jnp.float32),
            pltpu.VMEM(((N_CMP + 1) * CHUNK,), jnp.float32),
            pltpu.VMEM((N_CMP * MAXB * 256,), jnp.int32),
            pltpu.VMEM((288,), jnp.int32),
            pltpu.SemaphoreType.DMA,
            pltpu.SemaphoreType.DMA,
            pltpu.SemaphoreType.DMA,
            pltpu.SemaphoreType.DMA,
        ],
    )(_sc_body)


def kernel(x, mask):
    ext2, meta = _mask_prep(mask)
    return _sc_call()(x, ext2, meta)


# speedup vs baseline: 3.5521x; 2.9645x over previous
import functools

import jax
import jax.numpy as jnp
from jax import lax
from jax.experimental import pallas as pl
from jax.experimental.pallas import tpu as pltpu
from jax.experimental.pallas import tpu_sc as plsc

N_PTS = 131072
N_NODES = 64
N_CMP = 16
NW = 32
PTS_PER_W = N_PTS // NW
CHUNK = 512
NPAIR = PTS_PER_W // (2 * CHUNK)
MAXB = N_NODES // 16
PTILES = CHUNK // 128
XCH = CHUNK * N_NODES
OCH = CHUNK * N_CMP
NROW = 1024 * 1024

_NEG_INF = float("-inf")


def _mask_prep(mask):
    c_ids = jnp.arange(N_CMP, dtype=jnp.int32)
    n_ids = jnp.arange(N_NODES, dtype=jnp.int32)
    eq = jnp.all(mask[:, :, None] == mask[:, None, :], axis=0)
    first = jnp.argmax(eq, axis=0).astype(jnp.int32)
    is_rep = first == c_ids
    pos = jnp.cumsum(is_rep.astype(jnp.int32)) - 1
    num_d = jnp.sum(is_rep.astype(jnp.int32))
    posmap = pos[first]
    hasany = jnp.any(mask, axis=0)
    posmap2 = jnp.where(hasany, posmap, N_CMP).astype(jnp.int32)
    rep_ids = jnp.zeros((N_CMP,), jnp.int32).at[
        jnp.where(is_rep, pos, N_CMP)
    ].set(c_ids, mode="drop")
    m_rep = mask.T[rep_ids]
    cnt = jnp.sum(m_rep.astype(jnp.int32), axis=1)
    key = jnp.where(m_rep, 0, 1) * N_NODES + n_ids[None, :]
    order = jnp.argsort(key, axis=1).astype(jnp.int32)
    jidx = jnp.minimum(n_ids[None, :], jnp.maximum(cnt[:, None] - 1, 0))
    nlist = jnp.take_along_axis(order, jidx, axis=1)
    noff = (nlist // 8) * (CHUNK * 8) + (nlist % 8) * 128
    td = jnp.maximum((cnt + 15) // 16, 1).astype(jnp.int32)
    blocks = noff.reshape(N_CMP, MAXB, 16)
    s_ids = jnp.arange(16, dtype=jnp.int32)
    rot = (s_ids[:, None] + s_ids[None, :]) % 16
    ext2 = blocks[:, :, rot].reshape(-1)
    meta = jnp.concatenate([
        jnp.repeat(posmap2 * CHUNK, 16),
        jnp.full((16,), num_d, jnp.int32),
        jnp.repeat(td, 16),
    ])
    return ext2, meta


def _sc_body(x_hbm, ext_hbm, meta_hbm, out_hbm,
             xv, ov, resv, extv, metav,
             sem_in0, sem_in1, sem_out0, sem_out1):
    wid = lax.axis_index("s") * 2 + lax.axis_index("c")
    base_tile = wid * (PTS_PER_W // 128)
    iota = lax.iota(jnp.int32, 16)

    pltpu.sync_copy(ext_hbm, extv)
    pltpu.sync_copy(meta_hbm, metav)
    num_d = metav[pl.ds(256, 16)][0]
    rowbase = [metav[pl.ds(c * 16, 16)][0] for c in range(N_CMP)]

    def _zero_g(g, c):
        resv[pl.ds(N_CMP * CHUNK + g * 16, 16)] = jnp.zeros((16,), jnp.float32)
        return c
    lax.fori_loop(0, CHUNK // 16, _zero_g, 0)

    def compute_chunk(kchunk, buf):
        bufbase = buf * XCH

        def per_d(d, c):
            tdd = metav[pl.ds(272 + d * 16, 16)][0]

            def init_g(g, c2):
                resv[pl.ds(d * CHUNK + g * 16, 16)] = jnp.full(
                    (16,), _NEG_INF, jnp.float32)
                return c2
            lax.fori_loop(0, CHUNK // 16, init_g, 0)

            def per_block(bi, c2):
                eb = (d * MAXB + bi) * 256
                svs = [extv[pl.ds(eb + s * 16, 16)] for s in range(16)]

                def per_g(g, c3):
                    gg = g * 2
                    off = d * CHUNK + gg * 16
                    b0 = bufbase + (gg // 8) * 1024 + (gg % 8) * 16
                    b1 = bufbase + ((gg + 1) // 8) * 1024 + ((gg + 1) % 8) * 16
                    acc0 = resv[pl.ds(off, 16)]
                    acc1 = resv[pl.ds(off + 16, 16)]
                    ptv0 = b0 + iota
                    ptv1 = b1 + iota
                    for s in range(16):
                        acc0 = jnp.maximum(
                            acc0, plsc.load_gather(xv, [ptv0 + svs[s]]))
                        acc1 = jnp.maximum(
                            acc1, plsc.load_gather(xv, [ptv1 + svs[s]]))
                    resv[pl.ds(off, 16)] = acc0
                    resv[pl.ds(off + 16, 16)] = acc1
                    return c3
                lax.fori_loop(0, CHUNK // 32, per_g, 0)
                return c2
            lax.fori_loop(0, tdd, per_block, 0)
            return c
        lax.fori_loop(0, num_d, per_d, 0)

        bufo = buf * OCH
        for c in range(N_CMP):
            dbase = bufo + (c // 8) * (PTILES * 1024) + (c % 8) * 128
            rb = rowbase[c]

            def cp_tile(pt, c2, dbase=dbase, rb=rb):
                src = rb + pt * 128
                dst = dbase + pt * 1024
                for j in range(8):
                    ov[pl.ds(dst + j * 16, 16)] = resv[pl.ds(src + j * 16, 16)]
                return c2
            lax.fori_loop(0, PTILES, cp_tile, 0)

    def in_cpys(kchunk, buf, sem):
        t0 = (base_tile + kchunk * PTILES) * 1024
        return [
            pltpu.make_async_copy(
                x_hbm.at[pl.ds(nt * NROW + t0, PTILES * 1024)],
                xv.at[pl.ds(buf * XCH + nt * (PTILES * 1024), PTILES * 1024)],
                sem)
            for nt in range(8)
        ]

    def out_cpys(kchunk, buf, sem):
        t0 = (base_tile + kchunk * PTILES) * 1024
        return [
            pltpu.make_async_copy(
                ov.at[pl.ds(buf * OCH + ct * (PTILES * 1024), PTILES * 1024)],
                out_hbm.at[pl.ds(ct * NROW + t0, PTILES * 1024)],
                sem)
            for ct in range(2)
        ]

    def start(cpys):
        for cp in cpys:
            cp.start()

    def wait(cpys):
        for cp in cpys:
            cp.wait()

    start(in_cpys(0, 0, sem_in0))

    def per_pair(j, c):
        k0 = 2 * j
        wait(in_cpys(k0, 0, sem_in0))
        start(in_cpys(k0 + 1, 1, sem_in1))

        @pl.when(j > 0)
        def _():
            wait(out_cpys(k0 - 2, 0, sem_out0))
        compute_chunk(k0, 0)
        start(out_cpys(k0, 0, sem_out0))

        wait(in_cpys(k0 + 1, 1, sem_in1))

        @pl.when(j < NPAIR - 1)
        def _():
            start(in_cpys(k0 + 2, 0, sem_in0))

        @pl.when(j > 0)
        def _():
            wait(out_cpys(k0 - 1, 1, sem_out1))
        compute_chunk(k0 + 1, 1)
        start(out_cpys(k0 + 1, 1, sem_out1))
        return c

    lax.fori_loop(0, NPAIR, per_pair, 0)
    wait(out_cpys(2 * NPAIR - 2, 0, sem_out0))
    wait(out_cpys(2 * NPAIR - 1, 1, sem_out1))


@functools.cache
def _sc_call():
    return functools.partial(
        pl.kernel,
        out_type=jax.ShapeDtypeStruct((N_PTS * N_CMP,), jnp.float32),
        mesh=plsc.VectorSubcoreMesh(core_axis_name="c", subcore_axis_name="s"),
        compiler_params=pltpu.CompilerParams(
            needs_layout_passes=False, use_tc_tiling_on_sc=False),
        scratch_types=[
            pltpu.VMEM((2 * XCH,), jnp.float32),
            pltpu.VMEM((2 * OCH,), jnp.float32),
            pltpu.VMEM(((N_CMP + 1) * CHUNK,), jnp.float32),
            pltpu.VMEM((N_CMP * MAXB * 256,), jnp.int32),
            pltpu.VMEM((528,), jnp.int32),
            pltpu.SemaphoreType.DMA,
            pltpu.SemaphoreType.DMA,
            pltpu.SemaphoreType.DMA,
            pltpu.SemaphoreType.DMA,
        ],
    )(_sc_body)


def kernel(x, mask):
    ext2, meta = _mask_prep(mask)
    x_lin = (x.T.reshape(8, 8, 1024, 128).transpose(0, 2, 1, 3).reshape(-1))
    out_flat = _sc_call()(x_lin, ext2, meta)
    return (out_flat.reshape(2, 1024, 8, 128).transpose(0, 2, 1, 3)
            .reshape(16, 131072).T)
